# baseline (speedup 1.0000x reference)
import functools

import jax
import jax.numpy as jnp
import numpy as np
from jax import lax
from jax.experimental import pallas as pl
from jax.experimental.pallas import tpu as pltpu
from jax.experimental.pallas import tpu_sc as plsc

B, C, H, W = 32, 384, 24, 24
NPOS = B * H * W
P = 0.1
NUM_ZERO = int(P * C)

_PERM = np.asarray(
    jax.random.permutation(jax.random.key(42), C - 1)[:NUM_ZERO]
).astype(np.int32)

_SCALE = np.ones((C,), dtype=np.float32)
_SCALE[_PERM] = 1e-8
_GROUPS = [
    (g, _SCALE[g * 16 : (g + 1) * 16].copy())
    for g in range(C // 16)
    if (_SCALE[g * 16 : (g + 1) * 16] != 1.0).any()
]

NC, NS = 2, 16
NW = NC * NS
ROWS_PER_W = NPOS // NW

CHUNK_ROWS = 48
NCH = ROWS_PER_W // CHUNK_ROWS
NBUF = 6
LOOK = 3

_mesh = plsc.VectorSubcoreMesh(
    core_axis_name="c", subcore_axis_name="s", num_cores=NC, num_subcores=NS
)


@functools.partial(
    pl.kernel,
    out_type=jax.ShapeDtypeStruct((NPOS, C), jnp.float32),
    mesh=_mesh,
    compiler_params=pltpu.CompilerParams(needs_layout_passes=False),
    scratch_types=[
        [pltpu.VMEM((CHUNK_ROWS, C), jnp.float32) for _ in range(NBUF)],
        pltpu.SemaphoreType.DMA,
        pltpu.SemaphoreType.DMA,
    ],
)
def _sc_scale(x_hbm, out_hbm, bufs, isem, osem):
    wid = lax.axis_index("s") * NC + lax.axis_index("c")
    r0 = wid * ROWS_PER_W

    def start_in(j):
        return pltpu.async_copy(
            x_hbm.at[pl.ds(r0 + j * CHUNK_ROWS, CHUNK_ROWS), :],
            bufs[j % NBUF],
            isem,
        )

    def start_out(j):
        return pltpu.async_copy(
            bufs[j % NBUF],
            out_hbm.at[pl.ds(r0 + j * CHUNK_ROWS, CHUNK_ROWS), :],
            osem,
        )

    ins, outs = {}, {}
    for j in range(LOOK):
        ins[j] = start_in(j)
    lanes = lax.iota(jnp.int32, 16)
    ones_v = jnp.full((16,), 1.0, dtype=jnp.float32)
    small_v = jnp.full((16,), 1e-8, dtype=jnp.float32)
    scale_vecs = []
    for g, vec in _GROUPS:
        sv = ones_v
        for lane in np.nonzero(vec != 1.0)[0]:
            sv = jnp.where(lanes == int(lane), small_v, sv)
        scale_vecs.append((g, sv))
    for i in range(NCH):
        ins[i].wait()
        buf = bufs[i % NBUF]

        @plsc.parallel_loop(0, CHUNK_ROWS, step=1, unroll=4)
        def row_body(r, buf=buf):
            for g, vec in scale_vecs:
                buf[r, pl.ds(g * 16, 16)] = buf[r, pl.ds(g * 16, 16)] * vec
        outs[i] = start_out(i)
        j = i + LOOK
        if j < NCH:
            if j - NBUF >= 0:
                outs[j - NBUF].wait()
            ins[j] = start_in(j)
    for i in range(NCH - NBUF, NCH):
        outs[i].wait()


def kernel(x):
    x2 = x.transpose(0, 2, 3, 1).reshape(NPOS, C)
    out = _sc_scale(x2)
    return out.reshape(B, H, W, C).transpose(0, 3, 1, 2)


# speedup vs baseline: 1.0133x; 1.0133x over previous
import functools

import jax
import jax.numpy as jnp
import numpy as np
from jax import lax
from jax.experimental import pallas as pl
from jax.experimental.pallas import tpu as pltpu
from jax.experimental.pallas import tpu_sc as plsc

B, C, H, W = 32, 384, 24, 24
NPOS = B * H * W
P = 0.1
NUM_ZERO = int(P * C)

_PERM = np.asarray(
    jax.random.permutation(jax.random.key(42), C - 1)[:NUM_ZERO]
).astype(np.int32)

_SCALE = np.ones((C,), dtype=np.float32)
_SCALE[_PERM] = 1e-8
_GROUPS = [
    (g, _SCALE[g * 16 : (g + 1) * 16].copy())
    for g in range(C // 16)
    if (_SCALE[g * 16 : (g + 1) * 16] != 1.0).any()
]

NC, NS = 2, 16
NW = NC * NS
ROWS_PER_W = NPOS // NW

CHUNK_ROWS = 48
NCH = ROWS_PER_W // CHUNK_ROWS
NBUF = 6
LOOK = 4

_mesh = plsc.VectorSubcoreMesh(
    core_axis_name="c", subcore_axis_name="s", num_cores=NC, num_subcores=NS
)


@functools.partial(
    pl.kernel,
    out_type=jax.ShapeDtypeStruct((NPOS, C), jnp.float32),
    mesh=_mesh,
    compiler_params=pltpu.CompilerParams(needs_layout_passes=False),
    scratch_types=[
        [pltpu.VMEM((CHUNK_ROWS, C), jnp.float32) for _ in range(NBUF)],
        pltpu.SemaphoreType.DMA,
        pltpu.SemaphoreType.DMA,
    ],
)
def _sc_scale(x_hbm, out_hbm, bufs, isem, osem):
    wid = lax.axis_index("s") * NC + lax.axis_index("c")
    r0 = wid * ROWS_PER_W

    def start_in(j):
        return pltpu.async_copy(
            x_hbm.at[pl.ds(r0 + j * CHUNK_ROWS, CHUNK_ROWS), :],
            bufs[j % NBUF],
            isem,
        )

    def start_out(j):
        return pltpu.async_copy(
            bufs[j % NBUF],
            out_hbm.at[pl.ds(r0 + j * CHUNK_ROWS, CHUNK_ROWS), :],
            osem,
        )

    ins, outs = {}, {}
    for j in range(LOOK):
        ins[j] = start_in(j)
    lanes = lax.iota(jnp.int32, 16)
    ones_v = jnp.full((16,), 1.0, dtype=jnp.float32)
    small_v = jnp.full((16,), 1e-8, dtype=jnp.float32)
    scale_vecs = []
    for g, vec in _GROUPS:
        sv = ones_v
        for lane in np.nonzero(vec != 1.0)[0]:
            sv = jnp.where(lanes == int(lane), small_v, sv)
        scale_vecs.append((g, sv))
    for i in range(NCH):
        ins[i].wait()
        buf = bufs[i % NBUF]

        @plsc.parallel_loop(0, CHUNK_ROWS, step=1, unroll=2)
        def row_body(r, buf=buf):
            for g, vec in scale_vecs:
                buf[r, pl.ds(g * 16, 16)] = buf[r, pl.ds(g * 16, 16)] * vec
        outs[i] = start_out(i)
        j = i + LOOK
        if j < NCH:
            if j - NBUF >= 0:
                outs[j - NBUF].wait()
            ins[j] = start_in(j)
    for i in range(NCH - NBUF, NCH):
        outs[i].wait()


def kernel(x):
    x2 = x.transpose(0, 2, 3, 1).reshape(NPOS, C)
    out = _sc_scale(x2)
    return out.reshape(B, H, W, C).transpose(0, 3, 1, 2)


# speedup vs baseline: 1.0431x; 1.0294x over previous
import functools

import jax
import jax.numpy as jnp
import numpy as np
from jax import lax
from jax.experimental import pallas as pl
from jax.experimental.pallas import tpu as pltpu
from jax.experimental.pallas import tpu_sc as plsc

B, C, H, W = 32, 384, 24, 24
NPOS = B * H * W
P = 0.1
NUM_ZERO = int(P * C)

_PERM = np.asarray(
    jax.random.permutation(jax.random.key(42), C - 1)[:NUM_ZERO]
).astype(np.int32)

_SCALE = np.ones((C,), dtype=np.float32)
_SCALE[_PERM] = 1e-8
_GROUPS = [
    (g, _SCALE[g * 16 : (g + 1) * 16].copy())
    for g in range(C // 16)
    if (_SCALE[g * 16 : (g + 1) * 16] != 1.0).any()
]

NC, NS = 2, 16
NW = NC * NS
ROWS_PER_W = NPOS // NW

CHUNK_ROWS = 64
NCH = ROWS_PER_W // CHUNK_ROWS
NBUF = 5
LOOK = 3

_mesh = plsc.VectorSubcoreMesh(
    core_axis_name="c", subcore_axis_name="s", num_cores=NC, num_subcores=NS
)


@functools.partial(
    pl.kernel,
    out_type=jax.ShapeDtypeStruct((NPOS, C), jnp.float32),
    mesh=_mesh,
    compiler_params=pltpu.CompilerParams(needs_layout_passes=False),
    scratch_types=[
        [pltpu.VMEM((CHUNK_ROWS, C), jnp.float32) for _ in range(NBUF)],
        pltpu.SemaphoreType.DMA,
        pltpu.SemaphoreType.DMA,
    ],
)
def _sc_scale(x_hbm, out_hbm, bufs, isem, osem):
    wid = lax.axis_index("s") * NC + lax.axis_index("c")
    r0 = wid * ROWS_PER_W

    def start_in(j):
        return pltpu.async_copy(
            x_hbm.at[pl.ds(r0 + j * CHUNK_ROWS, CHUNK_ROWS), :],
            bufs[j % NBUF],
            isem,
        )

    def start_out(j):
        return pltpu.async_copy(
            bufs[j % NBUF],
            out_hbm.at[pl.ds(r0 + j * CHUNK_ROWS, CHUNK_ROWS), :],
            osem,
        )

    ins, outs = {}, {}
    for j in range(LOOK):
        ins[j] = start_in(j)
    lanes = lax.iota(jnp.int32, 16)
    ones_v = jnp.full((16,), 1.0, dtype=jnp.float32)
    small_v = jnp.full((16,), 1e-8, dtype=jnp.float32)
    scale_vecs = []
    for g, vec in _GROUPS:
        sv = ones_v
        for lane in np.nonzero(vec != 1.0)[0]:
            sv = jnp.where(lanes == int(lane), small_v, sv)
        scale_vecs.append((g, sv))
    for i in range(NCH):
        ins[i].wait()
        buf = bufs[i % NBUF]

        @plsc.parallel_loop(0, CHUNK_ROWS, step=1, unroll=2)
        def row_body(r, buf=buf):
            for g, vec in scale_vecs:
                buf[r, pl.ds(g * 16, 16)] = buf[r, pl.ds(g * 16, 16)] * vec
        outs[i] = start_out(i)
        j = i + LOOK
        if j < NCH:
            if j - NBUF >= 0:
                outs[j - NBUF].wait()
            ins[j] = start_in(j)
    for i in range(NCH - NBUF, NCH):
        outs[i].wait()


def kernel(x):
    x2 = x.transpose(0, 2, 3, 1).reshape(NPOS, C)
    out = _sc_scale(x2)
    return out.reshape(B, H, W, C).transpose(0, 3, 1, 2)


# speedup vs baseline: 1.0545x; 1.0109x over previous
import functools

import jax
import jax.numpy as jnp
import numpy as np
from jax import lax
from jax.experimental import pallas as pl
from jax.experimental.pallas import tpu as pltpu
from jax.experimental.pallas import tpu_sc as plsc

B, C, H, W = 32, 384, 24, 24
NPOS = B * H * W
P = 0.1
NUM_ZERO = int(P * C)

_PERM = np.asarray(
    jax.random.permutation(jax.random.key(42), C - 1)[:NUM_ZERO]
).astype(np.int32)

_SCALE = np.ones((C,), dtype=np.float32)
_SCALE[_PERM] = 1e-8
_GROUPS = [
    (g, _SCALE[g * 16 : (g + 1) * 16].copy())
    for g in range(C // 16)
    if (_SCALE[g * 16 : (g + 1) * 16] != 1.0).any()
]

NC, NS = 2, 16
NW = NC * NS
ROWS_PER_W = NPOS // NW

CHUNK_ROWS = 96
NCH = ROWS_PER_W // CHUNK_ROWS
NBUF = 3
LOOK = 2

_mesh = plsc.VectorSubcoreMesh(
    core_axis_name="c", subcore_axis_name="s", num_cores=NC, num_subcores=NS
)


@functools.partial(
    pl.kernel,
    out_type=jax.ShapeDtypeStruct((NPOS, C), jnp.float32),
    mesh=_mesh,
    compiler_params=pltpu.CompilerParams(needs_layout_passes=False),
    scratch_types=[
        [pltpu.VMEM((CHUNK_ROWS, C), jnp.float32) for _ in range(NBUF)],
        pltpu.SemaphoreType.DMA,
        pltpu.SemaphoreType.DMA,
    ],
)
def _sc_scale(x_hbm, out_hbm, bufs, isem, osem):
    wid = lax.axis_index("s") * NC + lax.axis_index("c")
    r0 = wid * ROWS_PER_W

    def start_in(j):
        return pltpu.async_copy(
            x_hbm.at[pl.ds(r0 + j * CHUNK_ROWS, CHUNK_ROWS), :],
            bufs[j % NBUF],
            isem,
        )

    def start_out(j):
        return pltpu.async_copy(
            bufs[j % NBUF],
            out_hbm.at[pl.ds(r0 + j * CHUNK_ROWS, CHUNK_ROWS), :],
            osem,
        )

    ins, outs = {}, {}
    for j in range(LOOK):
        ins[j] = start_in(j)
    lanes = lax.iota(jnp.int32, 16)
    ones_v = jnp.full((16,), 1.0, dtype=jnp.float32)
    small_v = jnp.full((16,), 1e-8, dtype=jnp.float32)
    scale_vecs = []
    for g, vec in _GROUPS:
        sv = ones_v
        for lane in np.nonzero(vec != 1.0)[0]:
            sv = jnp.where(lanes == int(lane), small_v, sv)
        scale_vecs.append((g, sv))
    for i in range(NCH):
        ins[i].wait()
        buf = bufs[i % NBUF]

        @plsc.parallel_loop(0, CHUNK_ROWS, step=1, unroll=2)
        def row_body(r, buf=buf):
            for g, vec in scale_vecs:
                buf[r, pl.ds(g * 16, 16)] = buf[r, pl.ds(g * 16, 16)] * vec
        outs[i] = start_out(i)
        j = i + LOOK
        if j < NCH:
            if j - NBUF >= 0:
                outs[j - NBUF].wait()
            ins[j] = start_in(j)
    for i in range(NCH - NBUF, NCH):
        outs[i].wait()


def kernel(x):
    x2 = x.transpose(0, 2, 3, 1).reshape(NPOS, C)
    out = _sc_scale(x2)
    return out.reshape(B, H, W, C).transpose(0, 3, 1, 2)
